# baseline (speedup 1.0000x reference)
import functools

import jax
import jax.numpy as jnp
from jax import lax
from jax.experimental import pallas as pl
from jax.experimental.pallas import tpu as pltpu
import jax.experimental.pallas.tpu_sc as plsc

_NC = 2
_NS = 16
_NW = _NC * _NS
_CHUNK = 128


@functools.cache
def _make_gather(B, D):
    bpw = B // _NW
    nch = bpw // _CHUNK
    mesh = plsc.VectorSubcoreMesh(core_axis_name="c", subcore_axis_name="s")

    @functools.partial(
        pl.kernel,
        out_type=(
            jax.ShapeDtypeStruct((B, D), jnp.float32),
            jax.ShapeDtypeStruct((B, D), jnp.float32),
        ),
        mesh=mesh,
        scratch_types=[
            pltpu.VMEM((nch, _CHUNK), jnp.int32),
            pltpu.VMEM((nch, _CHUNK), jnp.int32),
            pltpu.VMEM((bpw, D), jnp.float32),
            pltpu.VMEM((bpw, D), jnp.float32),
            pltpu.SemaphoreType.DMA,
        ],
        compiler_params=pltpu.CompilerParams(use_tc_tiling_on_sc=False),
    )
    def gather(user_hbm, item_hbm, ut_hbm, it_hbm, out_u, out_i,
               idx_u, idx_i, rows_u, rows_i, sem):
        wid = lax.axis_index("s") * _NC + lax.axis_index("c")
        base = wid * bpw
        pltpu.sync_copy(user_hbm.at[wid], idx_u)
        pltpu.sync_copy(item_hbm.at[wid], idx_i)
        copies = []
        for j in range(nch):
            copies.append(pltpu.async_copy(
                ut_hbm.at[idx_u.at[j]],
                rows_u.at[pl.ds(j * _CHUNK, _CHUNK)], sem))
            copies.append(pltpu.async_copy(
                it_hbm.at[idx_i.at[j]],
                rows_i.at[pl.ds(j * _CHUNK, _CHUNK)], sem))
        for c in copies:
            c.wait()
        pltpu.sync_copy(rows_u, out_u.at[pl.ds(base, bpw)])
        pltpu.sync_copy(rows_i, out_i.at[pl.ds(base, bpw)])

    return gather


def _mlp_body(u_ref, i_ref, w1a_ref, w1b_ref, b1_ref, w2_ref, b2_ref, o_ref):
    h = jnp.dot(u_ref[...], w1a_ref[...], preferred_element_type=jnp.float32)
    h = h + jnp.dot(i_ref[...], w1b_ref[...], preferred_element_type=jnp.float32)
    h = jnp.maximum(h + b1_ref[...], 0.0)
    z = jnp.dot(h, w2_ref[...], preferred_element_type=jnp.float32) + b2_ref[...]
    o_ref[...] = jax.nn.sigmoid(z)


@functools.cache
def _make_mlp(B, D, H, BM):
    grid = (B // BM,)
    return pl.pallas_call(
        _mlp_body,
        grid=grid,
        in_specs=[
            pl.BlockSpec((BM, D), lambda m: (m, 0)),
            pl.BlockSpec((BM, D), lambda m: (m, 0)),
            pl.BlockSpec((D, H), lambda m: (0, 0)),
            pl.BlockSpec((D, H), lambda m: (0, 0)),
            pl.BlockSpec((1, H), lambda m: (0, 0)),
            pl.BlockSpec((H, 1), lambda m: (0, 0)),
            pl.BlockSpec((1, 1), lambda m: (0, 0)),
        ],
        out_specs=pl.BlockSpec((BM, 1), lambda m: (m, 0)),
        out_shape=jax.ShapeDtypeStruct((B, 1), jnp.float32),
    )


def kernel(user, item, user_table, item_table, W1, b1, W2, b2):
    B = user.shape[0]
    D = user_table.shape[1]
    H = W1.shape[1]
    u_idx = user.astype(jnp.int32).reshape(_NW, -1, _CHUNK)
    i_idx = item.astype(jnp.int32).reshape(_NW, -1, _CHUNK)
    u_rows, i_rows = _make_gather(B, D)(u_idx, i_idx, user_table, item_table)
    w1a = W1[:D]
    w1b = W1[D:]
    out = _make_mlp(B, D, H, 2048)(
        u_rows, i_rows, w1a, w1b, b1.reshape(1, H), W2, b2.reshape(1, 1))
    return out


# speedup vs baseline: 1.2806x; 1.2806x over previous
import functools

import jax
import jax.numpy as jnp
from jax import lax
from jax.experimental import pallas as pl
from jax.experimental.pallas import tpu as pltpu
import jax.experimental.pallas.tpu_sc as plsc

_NC = 2
_NS = 16
_NW = _NC * _NS
_CHUNK = 128
_BS = 16384


@functools.cache
def _make_proj(V, VP, D, H, BS):
    nslab = VP // BS

    def body(t_hbm, w_ref, o_ref, buf, scr, sems):
        n = pl.program_id(0)
        h = pl.program_id(1)
        slot = lax.rem(n, 2)

        def start(step, s):
            off = pl.multiple_of(step * BS, 128)
            pltpu.make_async_copy(
                t_hbm.at[:, pl.ds(off, BS)], buf.at[s], sems.at[s]
            ).start()

        @pl.when(jnp.logical_and(n == 0, h == 0))
        def _():
            start(0, 0)

        @pl.when(h == 0)
        def _():
            @pl.when(n + 1 < nslab)
            def _():
                start(n + 1, 1 - slot)

            pltpu.make_async_copy(
                t_hbm.at[:, pl.ds(pl.multiple_of(n * BS, 128), BS)],
                buf.at[slot], sems.at[slot]
            ).wait()
            scr[...] = jnp.dot(w_ref[...], buf[slot],
                               preferred_element_type=jnp.float32)

        o_ref[...] = scr[pl.ds(h, 1), :].reshape(o_ref.shape)

    return pl.pallas_call(
        body,
        grid=(nslab, H),
        in_specs=[
            pl.BlockSpec(memory_space=pltpu.HBM),
            pl.BlockSpec((H, D), lambda n, h: (0, 0)),
        ],
        out_specs=pl.BlockSpec((BS,), lambda n, h: (h * nslab + n,)),
        out_shape=jax.ShapeDtypeStruct((H * VP,), jnp.float32),
        scratch_shapes=[
            pltpu.VMEM((2, D, BS), jnp.float32),
            pltpu.VMEM((H, BS), jnp.float32),
            pltpu.SemaphoreType.DMA((2,)),
        ],
    )


@functools.cache
def _make_gather_mlp(VP, VT, B, H):
    bpw = B // _NW
    nch = bpw // _CHUNK
    mesh = plsc.VectorSubcoreMesh(core_axis_name="c", subcore_axis_name="s")

    @functools.partial(
        pl.kernel,
        out_type=jax.ShapeDtypeStruct((B,), jnp.float32),
        mesh=mesh,
        scratch_types=[
            pltpu.VMEM((H * nch, _CHUNK), jnp.int32),
            pltpu.VMEM((H * nch, _CHUNK), jnp.int32),
            pltpu.VMEM((bpw,), jnp.int32),
            pltpu.VMEM((bpw,), jnp.int32),
            pltpu.VMEM((H, bpw), jnp.float32),
            pltpu.VMEM((H, bpw), jnp.float32),
            pltpu.VMEM((H, VT), jnp.float32),
            pltpu.VMEM((H, VT), jnp.float32),
            pltpu.VMEM((bpw,), jnp.float32),
            pltpu.VMEM((H,), jnp.float32),
            pltpu.VMEM((H,), jnp.float32),
            pltpu.VMEM((16,), jnp.float32),
            pltpu.SemaphoreType.DMA,
        ],
        compiler_params=pltpu.CompilerParams(
            use_tc_tiling_on_sc=False, needs_layout_passes=False),
    )
    def gather_mlp(uk_hbm, ik_hbm, ur_hbm, ir_hbm, pu_hbm, pi_hbm,
                   tu_hbm, ti_hbm, b1_hbm, w2_hbm, b2_hbm, out_hbm,
                   idx_u, idx_i, raw_u, raw_i, rows_u, rows_i,
                   tail_u, tail_i, out_v, b1_v, w2_v, b2_v, sem):
        wid = lax.axis_index("s") * _NC + lax.axis_index("c")
        base = wid * bpw
        pltpu.sync_copy(uk_hbm.at[wid], idx_u)
        pltpu.sync_copy(ik_hbm.at[wid], idx_i)
        pltpu.sync_copy(ur_hbm.at[wid], raw_u)
        pltpu.sync_copy(ir_hbm.at[wid], raw_i)
        pltpu.sync_copy(tu_hbm, tail_u)
        pltpu.sync_copy(ti_hbm, tail_i)
        pltpu.sync_copy(b1_hbm, b1_v)
        pltpu.sync_copy(w2_hbm, w2_v)
        pltpu.sync_copy(b2_hbm, b2_v)
        copies = []
        for k in range(H):
            for j in range(nch):
                copies.append(pltpu.async_copy(
                    pu_hbm.at[idx_u.at[k * nch + j]],
                    rows_u.at[k, pl.ds(j * _CHUNK, _CHUNK)], sem))
                copies.append(pltpu.async_copy(
                    pi_hbm.at[idx_i.at[k * nch + j]],
                    rows_i.at[k, pl.ds(j * _CHUNK, _CHUNK)], sem))
        for c in copies:
            c.wait()
        b1 = b1_v[...]
        w2 = w2_v[...]
        b2 = b2_v[...][0]

        def body(g, carry):
            ru = raw_u[pl.ds(g * 16, 16)]
            ri = raw_i[pl.ds(g * 16, 16)]
            tu_off = ru - VP
            ti_off = ri - VP
            um = tu_off >= 0
            im = ti_off >= 0
            tuc = jnp.maximum(tu_off, 0)
            tic = jnp.maximum(ti_off, 0)
            z = jnp.full((16,), b2, jnp.float32)
            for k in range(H):
                ks = jnp.full((16,), k, jnp.int32)
                pu = rows_u[k, pl.ds(g * 16, 16)]
                pi = rows_i[k, pl.ds(g * 16, 16)]
                pu = jnp.where(um, plsc.load_gather(tail_u, [ks, tuc]), pu)
                pi = jnp.where(im, plsc.load_gather(tail_i, [ks, tic]), pi)
                h = jnp.maximum(pu + pi + b1[k], 0.0)
                z = z + h * w2[k]
            out_v[pl.ds(g * 16, 16)] = 1.0 / (1.0 + jnp.exp(-z))
            return carry

        lax.fori_loop(0, bpw // 16, body, 0, unroll=2)
        pltpu.sync_copy(out_v, out_hbm.at[pl.ds(base, bpw)])

    return gather_mlp


def kernel(user, item, user_table, item_table, W1, b1, W2, b2):
    B = user.shape[0]
    V, D = user_table.shape
    H = W1.shape[1]
    VP = (V // 1024) * 1024
    VT = V - VP
    u_raw = user.astype(jnp.int32)
    i_raw = item.astype(jnp.int32)
    planes = jnp.arange(H, dtype=jnp.int32)[None, :, None] * VP
    u_pl = (jnp.minimum(u_raw, VP - 1).reshape(_NW, 1, -1) + planes)
    i_pl = (jnp.minimum(i_raw, VP - 1).reshape(_NW, 1, -1) + planes)
    u_pl = u_pl.reshape(_NW, -1, _CHUNK)
    i_pl = i_pl.reshape(_NW, -1, _CHUNK)
    W1u, W1i = W1[:D], W1[D:]
    proj = _make_proj(V, VP, D, H, _BS)
    pu = proj(user_table.T, W1u.T)
    pi = proj(item_table.T, W1i.T)
    tail_u = (user_table[VP:] @ W1u).T
    tail_i = (item_table[VP:] @ W1i).T
    out = _make_gather_mlp(VP, VT, B, H)(
        u_pl, i_pl, u_raw.reshape(_NW, -1), i_raw.reshape(_NW, -1),
        pu, pi, tail_u, tail_i, b1, W2.reshape(H), jnp.pad(b2, (0, 15)))
    return out.reshape(B, 1)


# speedup vs baseline: 4.2408x; 3.3116x over previous
import functools

import jax
import jax.numpy as jnp
from jax import lax
from jax.experimental import pallas as pl
from jax.experimental.pallas import tpu as pltpu
import jax.experimental.pallas.tpu_sc as plsc

_NC = 2
_NS = 16
_NW = _NC * _NS
_CHUNK = 128
_BS = 16384


@functools.cache
def _make_proj(V, VP, D, H, BS):
    nslab = VP // BS

    def body(t_hbm, w_ref, *rest):
        outs = rest[:H]
        buf, sems = rest[H], rest[H + 1]
        n = pl.program_id(0)
        slot = lax.rem(n, 2)

        def start(step, s):
            off = pl.multiple_of(step * BS, 128)
            pltpu.make_async_copy(
                t_hbm.at[:, pl.ds(off, BS)], buf.at[s], sems.at[s]
            ).start()

        @pl.when(n == 0)
        def _():
            start(0, 0)

        @pl.when(n + 1 < nslab)
        def _():
            start(n + 1, 1 - slot)

        pltpu.make_async_copy(
            t_hbm.at[:, pl.ds(pl.multiple_of(n * BS, 128), BS)],
            buf.at[slot], sems.at[slot]
        ).wait()
        p = jnp.dot(w_ref[...], buf[slot],
                    preferred_element_type=jnp.float32)
        for h in range(H):
            outs[h][...] = p[h]

    return pl.pallas_call(
        body,
        grid=(nslab,),
        in_specs=[
            pl.BlockSpec(memory_space=pltpu.HBM),
            pl.BlockSpec((H, D), lambda n: (0, 0)),
        ],
        out_specs=[pl.BlockSpec((BS,), lambda n: (n,)) for _ in range(H)],
        out_shape=[jax.ShapeDtypeStruct((VP,), jnp.float32)
                   for _ in range(H)],
        scratch_shapes=[
            pltpu.VMEM((2, D, BS), jnp.float32),
            pltpu.SemaphoreType.DMA((2,)),
        ],
    )


@functools.cache
def _make_gather_mlp(VP, VT, B, H):
    bpw = B // _NW
    nch = bpw // _CHUNK
    mesh = plsc.VectorSubcoreMesh(core_axis_name="c", subcore_axis_name="s")

    @functools.partial(
        pl.kernel,
        out_type=jax.ShapeDtypeStruct((B,), jnp.float32),
        mesh=mesh,
        scratch_types=[
            pltpu.VMEM((nch, _CHUNK), jnp.int32),
            pltpu.VMEM((nch, _CHUNK), jnp.int32),
            pltpu.VMEM((bpw,), jnp.int32),
            pltpu.VMEM((bpw,), jnp.int32),
            pltpu.VMEM((H, bpw), jnp.float32),
            pltpu.VMEM((H, bpw), jnp.float32),
            pltpu.VMEM((H, VT), jnp.float32),
            pltpu.VMEM((H, VT), jnp.float32),
            pltpu.VMEM((bpw,), jnp.float32),
            pltpu.VMEM((H,), jnp.float32),
            pltpu.VMEM((H,), jnp.float32),
            pltpu.VMEM((16,), jnp.float32),
            pltpu.SemaphoreType.DMA,
        ],
        compiler_params=pltpu.CompilerParams(
            use_tc_tiling_on_sc=False, needs_layout_passes=False),
    )
    def gather_mlp(uc_hbm, ic_hbm, ur_hbm, ir_hbm, *rest):
        pu_hbm = rest[:H]
        pi_hbm = rest[H:2 * H]
        (tu_hbm, ti_hbm, b1_hbm, w2_hbm, b2_hbm, out_hbm,
         idx_u, idx_i, raw_u, raw_i, rows_u, rows_i,
         tail_u, tail_i, out_v, b1_v, w2_v, b2_v, sem) = rest[2 * H:]
        wid = lax.axis_index("s") * _NC + lax.axis_index("c")
        base = wid * bpw
        pltpu.sync_copy(uc_hbm.at[wid], idx_u)
        pltpu.sync_copy(ic_hbm.at[wid], idx_i)
        pltpu.sync_copy(ur_hbm.at[wid], raw_u)
        pltpu.sync_copy(ir_hbm.at[wid], raw_i)
        pltpu.sync_copy(tu_hbm, tail_u)
        pltpu.sync_copy(ti_hbm, tail_i)
        pltpu.sync_copy(b1_hbm, b1_v)
        pltpu.sync_copy(w2_hbm, w2_v)
        pltpu.sync_copy(b2_hbm, b2_v)
        copies = []
        for k in range(H):
            for j in range(nch):
                copies.append(pltpu.async_copy(
                    pu_hbm[k].at[idx_u.at[j]],
                    rows_u.at[k, pl.ds(j * _CHUNK, _CHUNK)], sem))
                copies.append(pltpu.async_copy(
                    pi_hbm[k].at[idx_i.at[j]],
                    rows_i.at[k, pl.ds(j * _CHUNK, _CHUNK)], sem))
        for c in copies:
            c.wait()
        b1 = b1_v[...]
        w2 = w2_v[...]
        b2 = b2_v[...][0]

        def body(g, carry):
            ru = raw_u[pl.ds(g * 16, 16)]
            ri = raw_i[pl.ds(g * 16, 16)]
            tu_off = ru - VP
            ti_off = ri - VP
            um = tu_off >= 0
            im = ti_off >= 0
            tuc = jnp.maximum(tu_off, 0)
            tic = jnp.maximum(ti_off, 0)
            z = jnp.full((16,), b2, jnp.float32)
            for k in range(H):
                ks = jnp.full((16,), k, jnp.int32)
                pu = rows_u[k, pl.ds(g * 16, 16)]
                pi = rows_i[k, pl.ds(g * 16, 16)]
                pu = jnp.where(um, plsc.load_gather(tail_u, [ks, tuc]), pu)
                pi = jnp.where(im, plsc.load_gather(tail_i, [ks, tic]), pi)
                h = jnp.maximum(pu + pi + b1[k], 0.0)
                z = z + h * w2[k]
            out_v[pl.ds(g * 16, 16)] = 1.0 / (1.0 + jnp.exp(-z))
            return carry

        lax.fori_loop(0, bpw // 16, body, 0, unroll=2)
        pltpu.sync_copy(out_v, out_hbm.at[pl.ds(base, bpw)])

    return gather_mlp


def kernel(user, item, user_table, item_table, W1, b1, W2, b2):
    B = user.shape[0]
    V, D = user_table.shape
    H = W1.shape[1]
    VP = (V // 1024) * 1024
    VT = V - VP
    u_raw = user.astype(jnp.int32)
    i_raw = item.astype(jnp.int32)
    u_cl = jnp.minimum(u_raw, VP - 1).reshape(_NW, -1, _CHUNK)
    i_cl = jnp.minimum(i_raw, VP - 1).reshape(_NW, -1, _CHUNK)
    W1u, W1i = W1[:D], W1[D:]
    proj = _make_proj(V, VP, D, H, _BS)
    pu = proj(user_table.T, W1u.T)
    pi = proj(item_table.T, W1i.T)
    tail_u = (user_table[VP:] @ W1u).T
    tail_i = (item_table[VP:] @ W1i).T
    out = _make_gather_mlp(VP, VT, B, H)(
        u_cl, i_cl, u_raw.reshape(_NW, -1), i_raw.reshape(_NW, -1),
        *pu, *pi, tail_u, tail_i, b1, W2.reshape(H), jnp.pad(b2, (0, 15)))
    return out.reshape(B, 1)


# speedup vs baseline: 4.3811x; 1.0331x over previous
import functools

import jax
import jax.numpy as jnp
from jax import lax
from jax.experimental import pallas as pl
from jax.experimental.pallas import tpu as pltpu
import jax.experimental.pallas.tpu_sc as plsc

_NC = 2
_NS = 16
_NW = _NC * _NS
_CHUNK = 128
_BS = 16384


@functools.cache
def _make_proj(V, VP, D, H, BS):
    nslab = VP // BS

    def body(t_hbm, w_ref, *rest):
        outs = rest[:H]
        buf, sems = rest[H], rest[H + 1]
        n = pl.program_id(0)
        slot = lax.rem(n, 2)

        def start(step, s):
            off = pl.multiple_of(step * BS, 128)
            pltpu.make_async_copy(
                t_hbm.at[:, pl.ds(off, BS)], buf.at[s], sems.at[s]
            ).start()

        @pl.when(n == 0)
        def _():
            start(0, 0)

        @pl.when(n + 1 < nslab)
        def _():
            start(n + 1, 1 - slot)

        pltpu.make_async_copy(
            t_hbm.at[:, pl.ds(pl.multiple_of(n * BS, 128), BS)],
            buf.at[slot], sems.at[slot]
        ).wait()
        p = jnp.dot(w_ref[...], buf[slot],
                    preferred_element_type=jnp.float32)
        for h in range(H):
            outs[h][...] = p[h]

    return pl.pallas_call(
        body,
        grid=(nslab,),
        in_specs=[
            pl.BlockSpec(memory_space=pltpu.HBM),
            pl.BlockSpec((H, D), lambda n: (0, 0)),
        ],
        out_specs=[pl.BlockSpec((BS,), lambda n: (n,)) for _ in range(H)],
        out_shape=[jax.ShapeDtypeStruct((VP,), jnp.float32)
                   for _ in range(H)],
        scratch_shapes=[
            pltpu.VMEM((2, D, BS), jnp.float32),
            pltpu.SemaphoreType.DMA((2,)),
        ],
    )


@functools.cache
def _make_gather_u(VP, B, H):
    bpw = B // _NW
    nch = bpw // _CHUNK
    mesh = plsc.VectorSubcoreMesh(core_axis_name="c", subcore_axis_name="s")

    @functools.partial(
        pl.kernel,
        out_type=jax.ShapeDtypeStruct((H, B), jnp.float32),
        mesh=mesh,
        scratch_types=[
            pltpu.VMEM((nch, _CHUNK), jnp.int32),
            pltpu.VMEM((H, bpw), jnp.float32),
            pltpu.SemaphoreType.DMA,
        ],
        compiler_params=pltpu.CompilerParams(
            use_tc_tiling_on_sc=False, needs_layout_passes=False),
    )
    def gather_u(uc_hbm, *rest):
        pu_hbm = rest[:H]
        out_hbm, idx_u, rows_u, sem = rest[H:]
        wid = lax.axis_index("s") * _NC + lax.axis_index("c")
        base = wid * bpw
        pltpu.sync_copy(uc_hbm.at[wid], idx_u)
        copies = []
        for k in range(H):
            for j in range(nch):
                copies.append(pltpu.async_copy(
                    pu_hbm[k].at[idx_u.at[j]],
                    rows_u.at[k, pl.ds(j * _CHUNK, _CHUNK)], sem))
        for c in copies:
            c.wait()
        pltpu.sync_copy(rows_u, out_hbm.at[:, pl.ds(base, bpw)])

    return gather_u


@functools.cache
def _make_gather_mlp(VP, VT, B, H):
    bpw = B // _NW
    nch = bpw // _CHUNK
    mesh = plsc.VectorSubcoreMesh(core_axis_name="c", subcore_axis_name="s")

    @functools.partial(
        pl.kernel,
        out_type=jax.ShapeDtypeStruct((B,), jnp.float32),
        mesh=mesh,
        scratch_types=[
            pltpu.VMEM((nch, _CHUNK), jnp.int32),
            pltpu.VMEM((bpw,), jnp.int32),
            pltpu.VMEM((bpw,), jnp.int32),
            pltpu.VMEM((H, bpw), jnp.float32),
            pltpu.VMEM((H, bpw), jnp.float32),
            pltpu.VMEM((H, VT), jnp.float32),
            pltpu.VMEM((H, VT), jnp.float32),
            pltpu.VMEM((bpw,), jnp.float32),
            pltpu.VMEM((H,), jnp.float32),
            pltpu.VMEM((H,), jnp.float32),
            pltpu.VMEM((16,), jnp.float32),
            pltpu.SemaphoreType.DMA,
        ],
        compiler_params=pltpu.CompilerParams(
            use_tc_tiling_on_sc=False, needs_layout_passes=False),
    )
    def gather_mlp(ru_hbm, ic_hbm, ur_hbm, ir_hbm, *rest):
        pi_hbm = rest[:H]
        (tu_hbm, ti_hbm, b1_hbm, w2_hbm, b2_hbm, out_hbm,
         idx_i, raw_u, raw_i, rows_u, rows_i,
         tail_u, tail_i, out_v, b1_v, w2_v, b2_v, sem) = rest[H:]
        wid = lax.axis_index("s") * _NC + lax.axis_index("c")
        base = wid * bpw
        pltpu.sync_copy(ic_hbm.at[wid], idx_i)
        pltpu.sync_copy(ur_hbm.at[wid], raw_u)
        pltpu.sync_copy(ir_hbm.at[wid], raw_i)
        pltpu.sync_copy(tu_hbm, tail_u)
        pltpu.sync_copy(ti_hbm, tail_i)
        pltpu.sync_copy(b1_hbm, b1_v)
        pltpu.sync_copy(w2_hbm, w2_v)
        pltpu.sync_copy(b2_hbm, b2_v)
        copies = [pltpu.async_copy(
            ru_hbm.at[:, pl.ds(base, bpw)], rows_u, sem)]
        for k in range(H):
            for j in range(nch):
                copies.append(pltpu.async_copy(
                    pi_hbm[k].at[idx_i.at[j]],
                    rows_i.at[k, pl.ds(j * _CHUNK, _CHUNK)], sem))
        for c in copies:
            c.wait()
        b1 = b1_v[...]
        w2 = w2_v[...]
        b2 = b2_v[...][0]

        def body(g, carry):
            ru = raw_u[pl.ds(g * 16, 16)]
            ri = raw_i[pl.ds(g * 16, 16)]
            um = ru >= VP
            im = ri >= VP
            tuc = jnp.maximum(ru - VP, 0)
            tic = jnp.maximum(ri - VP, 0)
            z = jnp.full((16,), b2, jnp.float32)
            for k in range(H):
                ks = jnp.full((16,), k, jnp.int32)
                pu = rows_u[k, pl.ds(g * 16, 16)]
                pi = rows_i[k, pl.ds(g * 16, 16)]
                pu = jnp.where(um, plsc.load_gather(tail_u, [ks, tuc]), pu)
                pi = jnp.where(im, plsc.load_gather(tail_i, [ks, tic]), pi)
                h = jnp.maximum(pu + pi + b1[k], 0.0)
                z = z + h * w2[k]
            out_v[pl.ds(g * 16, 16)] = 1.0 / (1.0 + jnp.exp(-z))
            return carry

        lax.fori_loop(0, bpw // 16, body, 0, unroll=2)
        pltpu.sync_copy(out_v, out_hbm.at[pl.ds(base, bpw)])

    return gather_mlp


def kernel(user, item, user_table, item_table, W1, b1, W2, b2):
    B = user.shape[0]
    V, D = user_table.shape
    H = W1.shape[1]
    VP = (V // 1024) * 1024
    VT = V - VP
    u_raw = user.astype(jnp.int32)
    i_raw = item.astype(jnp.int32)
    u_cl = jnp.minimum(u_raw, VP - 1).reshape(_NW, -1, _CHUNK)
    i_cl = jnp.minimum(i_raw, VP - 1).reshape(_NW, -1, _CHUNK)
    W1u, W1i = W1[:D], W1[D:]
    proj = _make_proj(V, VP, D, H, _BS)
    pu = proj(user_table.T, W1u.T)
    pi = proj(item_table.T, W1i.T)
    tail_u = (user_table[VP:] @ W1u).T
    tail_i = (item_table[VP:] @ W1i).T
    ru = _make_gather_u(VP, B, H)(u_cl, *pu)
    out = _make_gather_mlp(VP, VT, B, H)(
        ru, i_cl, u_raw.reshape(_NW, -1), i_raw.reshape(_NW, -1),
        *pi, tail_u, tail_i, b1, W2.reshape(H), jnp.pad(b2, (0, 15)))
    return out.reshape(B, 1)


# speedup vs baseline: 4.6421x; 1.0596x over previous
import functools

import jax
import jax.numpy as jnp
from jax import lax
from jax.experimental import pallas as pl
from jax.experimental.pallas import tpu as pltpu
import jax.experimental.pallas.tpu_sc as plsc

_NC = 2
_NS = 16
_NW = _NC * _NS
_CHUNK = 128
_BS = 16384


@functools.cache
def _make_proj(V, VP, D, H, BS):
    nslab = VP // BS

    def body(t_hbm, w_ref, *rest):
        outs = rest[:H]
        buf, sems = rest[H], rest[H + 1]
        n = pl.program_id(0)
        slot = lax.rem(n, 2)

        def start(step, s):
            off = pl.multiple_of(step * BS, 128)
            pltpu.make_async_copy(
                t_hbm.at[:, pl.ds(off, BS)], buf.at[s], sems.at[s]
            ).start()

        @pl.when(n == 0)
        def _():
            start(0, 0)

        @pl.when(n + 1 < nslab)
        def _():
            start(n + 1, 1 - slot)

        pltpu.make_async_copy(
            t_hbm.at[:, pl.ds(pl.multiple_of(n * BS, 128), BS)],
            buf.at[slot], sems.at[slot]
        ).wait()
        p = jnp.dot(w_ref[...], buf[slot],
                    preferred_element_type=jnp.float32)
        for h in range(H):
            row = p[h]
            lo = lax.bitcast_convert_type(
                row[:BS // 2].astype(jnp.bfloat16), jnp.uint16
            ).astype(jnp.uint32)
            hi = lax.bitcast_convert_type(
                row[BS // 2:].astype(jnp.bfloat16), jnp.uint16
            ).astype(jnp.uint32)
            outs[h][...] = lax.bitcast_convert_type(
                lo | (hi << 16), jnp.float32)

    return pl.pallas_call(
        body,
        grid=(nslab,),
        in_specs=[
            pl.BlockSpec(memory_space=pltpu.HBM),
            pl.BlockSpec((H, D), lambda n: (0, 0)),
        ],
        out_specs=[pl.BlockSpec((BS // 2,), lambda n: (n,))
                   for _ in range(H)],
        out_shape=[jax.ShapeDtypeStruct((VP // 2,), jnp.float32)
                   for _ in range(H)],
        scratch_shapes=[
            pltpu.VMEM((2, D, BS), jnp.float32),
            pltpu.SemaphoreType.DMA((2,)),
        ],
    )


@functools.cache
def _make_gather_u(VP, B, H, BS):
    bpw = B // _NW
    nch = bpw // _CHUNK
    mesh = plsc.VectorSubcoreMesh(core_axis_name="c", subcore_axis_name="s")

    @functools.partial(
        pl.kernel,
        out_type=jax.ShapeDtypeStruct((H, B), jnp.float32),
        mesh=mesh,
        scratch_types=[
            pltpu.VMEM((nch, _CHUNK), jnp.int32),
            pltpu.VMEM((H, bpw), jnp.float32),
            pltpu.SemaphoreType.DMA,
        ],
        compiler_params=pltpu.CompilerParams(
            use_tc_tiling_on_sc=False, needs_layout_passes=False),
    )
    def gather_u(uc_hbm, *rest):
        pu_hbm = rest[:H]
        out_hbm, idx_u, rows_u, sem = rest[H:]
        wid = lax.axis_index("s") * _NC + lax.axis_index("c")
        base = wid * bpw
        pltpu.sync_copy(uc_hbm.at[wid], idx_u)
        copies = []
        for k in range(H):
            for j in range(nch):
                copies.append(pltpu.async_copy(
                    pu_hbm[k].at[idx_u.at[j]],
                    rows_u.at[k, pl.ds(j * _CHUNK, _CHUNK)], sem))
        for c in copies:
            c.wait()
        pltpu.sync_copy(rows_u, out_hbm.at[:, pl.ds(base, bpw)])

    return gather_u


@functools.cache
def _make_gather_mlp(VP, VT, B, H, BS):
    bpw = B // _NW
    nch = bpw // _CHUNK
    mesh = plsc.VectorSubcoreMesh(core_axis_name="c", subcore_axis_name="s")

    @functools.partial(
        pl.kernel,
        out_type=jax.ShapeDtypeStruct((B,), jnp.float32),
        mesh=mesh,
        scratch_types=[
            pltpu.VMEM((nch, _CHUNK), jnp.int32),
            pltpu.VMEM((bpw,), jnp.int32),
            pltpu.VMEM((bpw,), jnp.int32),
            pltpu.VMEM((H, bpw), jnp.float32),
            pltpu.VMEM((H, bpw), jnp.float32),
            pltpu.VMEM((H, VT), jnp.float32),
            pltpu.VMEM((H, VT), jnp.float32),
            pltpu.VMEM((bpw,), jnp.float32),
            pltpu.VMEM((H,), jnp.float32),
            pltpu.VMEM((H,), jnp.float32),
            pltpu.VMEM((16,), jnp.float32),
            pltpu.SemaphoreType.DMA,
        ],
        compiler_params=pltpu.CompilerParams(
            use_tc_tiling_on_sc=False, needs_layout_passes=False),
    )
    def gather_mlp(ru_hbm, ic_hbm, ur_hbm, ir_hbm, *rest):
        pi_hbm = rest[:H]
        (tu_hbm, ti_hbm, b1_hbm, w2_hbm, b2_hbm, out_hbm,
         idx_i, raw_u, raw_i, rows_u, rows_i,
         tail_u, tail_i, out_v, b1_v, w2_v, b2_v, sem) = rest[H:]
        wid = lax.axis_index("s") * _NC + lax.axis_index("c")
        base = wid * bpw
        pltpu.sync_copy(ic_hbm.at[wid], idx_i)
        pltpu.sync_copy(ur_hbm.at[wid], raw_u)
        pltpu.sync_copy(ir_hbm.at[wid], raw_i)
        pltpu.sync_copy(tu_hbm, tail_u)
        pltpu.sync_copy(ti_hbm, tail_i)
        pltpu.sync_copy(b1_hbm, b1_v)
        pltpu.sync_copy(w2_hbm, w2_v)
        pltpu.sync_copy(b2_hbm, b2_v)
        copies = [pltpu.async_copy(
            ru_hbm.at[:, pl.ds(base, bpw)], rows_u, sem)]
        for k in range(H):
            for j in range(nch):
                copies.append(pltpu.async_copy(
                    pi_hbm[k].at[idx_i.at[j]],
                    rows_i.at[k, pl.ds(j * _CHUNK, _CHUNK)], sem))
        for c in copies:
            c.wait()
        b1 = b1_v[...]
        w2 = w2_v[...]
        b2 = b2_v[...][0]

        def body(g, carry):
            ru = raw_u[pl.ds(g * 16, 16)]
            ri = raw_i[pl.ds(g * 16, 16)]
            um = ru >= VP
            im = ri >= VP
            tuc = jnp.maximum(ru - VP, 0)
            tic = jnp.maximum(ri - VP, 0)
            hu = (jnp.minimum(ru, VP - 1) & (BS - 1)) >= BS // 2
            hi_ = (jnp.minimum(ri, VP - 1) & (BS - 1)) >= BS // 2
            z = jnp.full((16,), b2, jnp.float32)
            for k in range(H):
                ks = jnp.full((16,), k, jnp.int32)
                pu_l, pu_h = plsc.unpack(
                    plsc.bitcast(rows_u[k, pl.ds(g * 16, 16)], jnp.bfloat16),
                    format=plsc.PackFormat.INTERLEAVED)
                pi_l, pi_h = plsc.unpack(
                    plsc.bitcast(rows_i[k, pl.ds(g * 16, 16)], jnp.bfloat16),
                    format=plsc.PackFormat.INTERLEAVED)
                pu = jnp.where(hu, pu_h, pu_l)
                pi = jnp.where(hi_, pi_h, pi_l)
                pu = jnp.where(um, plsc.load_gather(tail_u, [ks, tuc]), pu)
                pi = jnp.where(im, plsc.load_gather(tail_i, [ks, tic]), pi)
                h = jnp.maximum(pu + pi + b1[k], 0.0)
                z = z + h * w2[k]
            out_v[pl.ds(g * 16, 16)] = 1.0 / (1.0 + jnp.exp(-z))
            return carry

        lax.fori_loop(0, bpw // 16, body, 0, unroll=2)
        pltpu.sync_copy(out_v, out_hbm.at[pl.ds(base, bpw)])

    return gather_mlp


def kernel(user, item, user_table, item_table, W1, b1, W2, b2):
    B = user.shape[0]
    V, D = user_table.shape
    H = W1.shape[1]
    VP = (V // 1024) * 1024
    VT = V - VP
    u_raw = user.astype(jnp.int32)
    i_raw = item.astype(jnp.int32)
    def _jidx(raw):
        cl = jnp.minimum(raw, VP - 1)
        return (cl // _BS) * (_BS // 2) + (cl % (_BS // 2))
    u_cl = _jidx(u_raw).reshape(_NW, -1, _CHUNK)
    i_cl = _jidx(i_raw).reshape(_NW, -1, _CHUNK)
    W1u, W1i = W1[:D], W1[D:]
    proj = _make_proj(V, VP, D, H, _BS)
    pu = proj(user_table.T, W1u.T)
    pi = proj(item_table.T, W1i.T)
    tail_u = (user_table[VP:] @ W1u).T
    tail_i = (item_table[VP:] @ W1i).T
    ru = _make_gather_u(VP, B, H, _BS)(u_cl, *pu)
    out = _make_gather_mlp(VP, VT, B, H, _BS)(
        ru, i_cl, u_raw.reshape(_NW, -1), i_raw.reshape(_NW, -1),
        *pi, tail_u, tail_i, b1, W2.reshape(H), jnp.pad(b2, (0, 15)))
    return out.reshape(B, 1)
